# baseline (speedup 1.0000x reference)
import functools

import jax
import jax.numpy as jnp
from jax import lax
from jax.experimental import pallas as pl
from jax.experimental.pallas import tpu as pltpu
from jax.experimental.pallas import tpu_sc as plsc

_NUM_EMB = 1024
_DIM = 64
_ROWS = 18432
_HALF = _ROWS // 2
_TILE = 3072

_info = plsc.get_sparse_core_info()
_NC, _NS = _info.num_cores, _info.num_subcores
_NW = _NC * _NS
_CH = 96


def _dist_body(z_ref, cb_ref, idx_ref, loss_ref):
    i = pl.program_id(0)
    zt = z_ref[...]
    cb = cb_ref[...]
    m = lax.dot_general(zt, cb, (((1,), (1,)), ((), ())),
                        preferred_element_type=jnp.float32)
    z2 = jnp.sum(zt * zt, axis=1, keepdims=True)
    c2 = jnp.sum(cb * cb, axis=1)[None, :]
    d = (z2 + c2) - 2.0 * m
    dmin = jnp.min(d, axis=1, keepdims=True)
    j = lax.broadcasted_iota(jnp.int32, (1, _NUM_EMB), 1).astype(jnp.float32)
    idxf = jnp.min(jnp.where(d == dmin, j, jnp.float32(_NUM_EMB)), axis=1)
    idx_ref[...] = idxf.astype(jnp.int32)

    @pl.when(i == 0)
    def _():
        loss_ref[0, 0] = 0.0

    loss_ref[0, 0] += jnp.sum(dmin)


def _make_argmin(rows):
    grid = rows // _TILE
    return pl.pallas_call(
        _dist_body,
        grid=(grid,),
        in_specs=[
            pl.BlockSpec((_TILE, _DIM), lambda i: (i, 0)),
            pl.BlockSpec((_NUM_EMB, _DIM), lambda i: (0, 0)),
        ],
        out_specs=[
            pl.BlockSpec((_TILE,), lambda i: (i,)),
            pl.BlockSpec((1, 1), lambda i: (0, 0), memory_space=pltpu.SMEM),
        ],
        out_shape=[
            jax.ShapeDtypeStruct((rows,), jnp.int32),
            jax.ShapeDtypeStruct((1, 1), jnp.float32),
        ],
    )


_sc_mesh = plsc.VectorSubcoreMesh(core_axis_name="c", subcore_axis_name="s")


def _make_gather(rows):
    bpw = rows // _NW
    nch = bpw // _CH

    @functools.partial(
        pl.kernel,
        mesh=_sc_mesh,
        out_type=jax.ShapeDtypeStruct((rows, 2 * _DIM), jnp.float32),
        scratch_types=[
            pltpu.VMEM((bpw,), jnp.int32),
            pltpu.VMEM((bpw, 2 * _DIM), jnp.float32),
            pltpu.SemaphoreType.DMA,
        ],
    )
    def _sc_gather(cb_hbm, idx_hbm, out_hbm, idx_v, rows_v, sem):
        wid = lax.axis_index("s") * _NC + lax.axis_index("c")
        base = wid * bpw
        pltpu.sync_copy(idx_hbm.at[pl.ds(base, bpw)], idx_v)
        copies = []
        for j in range(nch):
            copies.append(
                pltpu.async_copy(
                    cb_hbm.at[idx_v.at[pl.ds(j * _CH, _CH)]],
                    rows_v.at[pl.ds(j * _CH, _CH)],
                    sem,
                ))
        for c in copies:
            c.wait()
        pltpu.sync_copy(rows_v, out_hbm.at[pl.ds(base, bpw)])

    return _sc_gather


_argmin_half = _make_argmin(_HALF)
_gather_half = _make_gather(_HALF)


def kernel(z, codebook):
    zz = z[0]
    z_flat = zz.reshape(-1, zz.shape[-1])
    cb_pad = jnp.pad(codebook, ((0, 0), (0, _DIM)))
    idx_a, ls_a = _argmin_half(z_flat[:_HALF], codebook)
    idx_b, ls_b = _argmin_half(z_flat[_HALF:], codebook)
    zq_a = _gather_half(cb_pad, idx_a)[:, :_DIM]
    zq_b = _gather_half(cb_pad, idx_b)[:, :_DIM]
    zq = jnp.concatenate([zq_a, zq_b], axis=0)
    m = (ls_a[0, 0] + ls_b[0, 0]) / (_ROWS * _DIM)
    vq_loss = m + 0.1 * m
    return zq.reshape(zz.shape), vq_loss


# speedup vs baseline: 1.1485x; 1.1485x over previous
import functools

import jax
import jax.numpy as jnp
from jax import lax
from jax.experimental import pallas as pl
from jax.experimental.pallas import tpu as pltpu
from jax.experimental.pallas import tpu_sc as plsc

_NUM_EMB = 1024
_DIM = 64
_ROWS = 18432
_HALF = _ROWS // 2
_TILE = 3072

_info = plsc.get_sparse_core_info()
_NC, _NS = _info.num_cores, _info.num_subcores
_NW = _NC * _NS
_CH = 96


def _dist_body(z_ref, cb_ref, idx_ref, loss_ref):
    i = pl.program_id(0)
    zt = z_ref[...]
    cb = cb_ref[...]
    m = lax.dot_general(zt, cb, (((1,), (1,)), ((), ())),
                        preferred_element_type=jnp.float32)
    z2 = jnp.sum(zt * zt, axis=1, keepdims=True)
    c2 = jnp.sum(cb * cb, axis=1)[None, :]
    jrow = lax.broadcasted_iota(jnp.int32, (1, 128), 1).astype(jnp.float32)
    run_min = None
    for c in range(_NUM_EMB // 128):
        mc = lax.slice(m, (0, c * 128), (_TILE, (c + 1) * 128))
        cc = lax.slice(c2, (0, c * 128), (1, (c + 1) * 128))
        dc = (z2 + cc) - 2.0 * mc
        jc = jrow + jnp.float32(128 * c)
        if run_min is None:
            run_min = dc
            run_arg = jnp.broadcast_to(jc, dc.shape)
        else:
            lt = dc < run_min
            run_min = jnp.where(lt, dc, run_min)
            run_arg = jnp.where(lt, jc, run_arg)
    gmin = jnp.min(run_min, axis=1, keepdims=True)
    idxf = jnp.min(
        jnp.where(run_min == gmin, run_arg, jnp.float32(_NUM_EMB)), axis=1)
    idx_ref[...] = idxf.astype(jnp.int32)

    @pl.when(i == 0)
    def _():
        loss_ref[0, 0] = 0.0

    loss_ref[0, 0] += jnp.sum(gmin)


def _make_argmin(rows):
    grid = rows // _TILE
    return pl.pallas_call(
        _dist_body,
        grid=(grid,),
        in_specs=[
            pl.BlockSpec((_TILE, _DIM), lambda i: (i, 0)),
            pl.BlockSpec((_NUM_EMB, _DIM), lambda i: (0, 0)),
        ],
        out_specs=[
            pl.BlockSpec((_TILE,), lambda i: (i,)),
            pl.BlockSpec((1, 1), lambda i: (0, 0), memory_space=pltpu.SMEM),
        ],
        out_shape=[
            jax.ShapeDtypeStruct((rows,), jnp.int32),
            jax.ShapeDtypeStruct((1, 1), jnp.float32),
        ],
    )


_sc_mesh = plsc.VectorSubcoreMesh(core_axis_name="c", subcore_axis_name="s")


def _make_gather(rows):
    bpw = rows // _NW
    nch = bpw // _CH

    @functools.partial(
        pl.kernel,
        mesh=_sc_mesh,
        out_type=jax.ShapeDtypeStruct((rows, 2 * _DIM), jnp.float32),
        scratch_types=[
            pltpu.VMEM((bpw,), jnp.int32),
            pltpu.VMEM((bpw, 2 * _DIM), jnp.float32),
            pltpu.SemaphoreType.DMA,
        ],
    )
    def _sc_gather(cb_hbm, idx_hbm, out_hbm, idx_v, rows_v, sem):
        wid = lax.axis_index("s") * _NC + lax.axis_index("c")
        base = wid * bpw
        pltpu.sync_copy(idx_hbm.at[pl.ds(base, bpw)], idx_v)
        copies = []
        for j in range(nch):
            copies.append(
                pltpu.async_copy(
                    cb_hbm.at[idx_v.at[pl.ds(j * _CH, _CH)]],
                    rows_v.at[pl.ds(j * _CH, _CH)],
                    sem,
                ))
        for c in copies:
            c.wait()
        pltpu.sync_copy(rows_v, out_hbm.at[pl.ds(base, bpw)])

    return _sc_gather


_argmin_full = _make_argmin(_ROWS)
_gather_full = _make_gather(_ROWS)


def kernel(z, codebook):
    zz = z[0]
    z_flat = zz.reshape(-1, zz.shape[-1])
    cb_pad = jnp.pad(codebook, ((0, 0), (0, _DIM)))
    idx, loss_sum = _argmin_full(z_flat, codebook)
    zq = _gather_full(cb_pad, idx)[:, :_DIM]
    m = loss_sum[0, 0] / (_ROWS * _DIM)
    vq_loss = m + 0.1 * m
    return zq.reshape(zz.shape), vq_loss


# speedup vs baseline: 1.3033x; 1.1348x over previous
import functools

import jax
import jax.numpy as jnp
from jax import lax
from jax.experimental import pallas as pl
from jax.experimental.pallas import tpu as pltpu
from jax.experimental.pallas import tpu_sc as plsc

_NUM_EMB = 1024
_DIM = 64
_ROWS = 18432
_HALF = _ROWS // 2
_TILE = 2048
_GRID = _ROWS // _TILE

_info = plsc.get_sparse_core_info()
_NC, _NS = _info.num_cores, _info.num_subcores
_NW = _NC * _NS
_CH = 96


def _dist_body(z_ref, cb_ref, idx_ref, loss_ref, m_buf, z2_buf):
    i = pl.program_id(0)
    cur = lax.rem(i, 2)
    prev = lax.rem(i + 1, 2)
    cb = cb_ref[...]
    c2 = jnp.sum(cb * cb, axis=1)[None, :]

    mp = m_buf[prev]
    z2p = z2_buf[prev]
    d = (z2p + c2) - 2.0 * mp
    dmin = jnp.min(d, axis=1, keepdims=True)
    j = lax.broadcasted_iota(jnp.int32, (1, _NUM_EMB), 1).astype(jnp.float32)
    idxf = jnp.min(jnp.where(d == dmin, j, jnp.float32(_NUM_EMB)), axis=1)
    idx_ref[...] = idxf.astype(jnp.int32)

    @pl.when(i == 0)
    def _():
        loss_ref[0, 0] = 0.0

    loss_ref[0, 0] += jnp.where(i > 0, jnp.sum(dmin), 0.0)

    zt = z_ref[...]
    m_buf[cur] = lax.dot_general(zt, cb, (((1,), (1,)), ((), ())),
                                 preferred_element_type=jnp.float32)
    z2_buf[cur] = jnp.sum(zt * zt, axis=1, keepdims=True)


def _make_argmin(rows):
    grid = rows // _TILE
    return pl.pallas_call(
        _dist_body,
        grid=(grid + 1,),
        in_specs=[
            pl.BlockSpec((_TILE, _DIM), lambda i: (jnp.minimum(i, grid - 1), 0)),
            pl.BlockSpec((_NUM_EMB, _DIM), lambda i: (0, 0)),
        ],
        out_specs=[
            pl.BlockSpec((_TILE,), lambda i: (jnp.maximum(i, 1) - 1,)),
            pl.BlockSpec((1, 1), lambda i: (0, 0), memory_space=pltpu.SMEM),
        ],
        out_shape=[
            jax.ShapeDtypeStruct((rows,), jnp.int32),
            jax.ShapeDtypeStruct((1, 1), jnp.float32),
        ],
        scratch_shapes=[
            pltpu.VMEM((2, _TILE, _NUM_EMB), jnp.float32),
            pltpu.VMEM((2, _TILE, 1), jnp.float32),
        ],
    )


_sc_mesh = plsc.VectorSubcoreMesh(core_axis_name="c", subcore_axis_name="s")


def _make_gather(rows):
    bpw = rows // _NW
    nch = bpw // _CH

    @functools.partial(
        pl.kernel,
        mesh=_sc_mesh,
        out_type=jax.ShapeDtypeStruct((rows, 2 * _DIM), jnp.float32),
        scratch_types=[
            pltpu.VMEM((bpw,), jnp.int32),
            pltpu.VMEM((bpw, 2 * _DIM), jnp.float32),
            pltpu.SemaphoreType.DMA,
        ],
    )
    def _sc_gather(cb_hbm, idx_hbm, out_hbm, idx_v, rows_v, sem):
        wid = lax.axis_index("s") * _NC + lax.axis_index("c")
        base = wid * bpw
        pltpu.sync_copy(idx_hbm.at[pl.ds(base, bpw)], idx_v)
        copies = []
        for j in range(nch):
            copies.append(
                pltpu.async_copy(
                    cb_hbm.at[idx_v.at[pl.ds(j * _CH, _CH)]],
                    rows_v.at[pl.ds(j * _CH, _CH)],
                    sem,
                ))
        for c in copies:
            c.wait()
        pltpu.sync_copy(rows_v, out_hbm.at[pl.ds(base, bpw)])

    return _sc_gather


_argmin_full = _make_argmin(_ROWS)
_gather_full = _make_gather(_ROWS)


def kernel(z, codebook):
    zz = z[0]
    z_flat = zz.reshape(-1, zz.shape[-1])
    cb_pad = jnp.pad(codebook, ((0, 0), (0, _DIM)))
    idx, loss_sum = _argmin_full(z_flat, codebook)
    zq = _gather_full(cb_pad, idx)[:, :_DIM]
    m = loss_sum[0, 0] / (_ROWS * _DIM)
    vq_loss = m + 0.1 * m
    return zq.reshape(zz.shape), vq_loss


# speedup vs baseline: 1.3045x; 1.0009x over previous
import functools

import jax
import jax.numpy as jnp
from jax import lax
from jax.experimental import pallas as pl
from jax.experimental.pallas import tpu as pltpu
from jax.experimental.pallas import tpu_sc as plsc

_NUM_EMB = 1024
_DIM = 64
_ROWS = 18432
_HALF = _ROWS // 2
_TILE = 2048
_GRID = _ROWS // _TILE

_info = plsc.get_sparse_core_info()
_NC, _NS = _info.num_cores, _info.num_subcores
_NW = _NC * _NS
_CH = 128


def _dist_body(z_ref, cb_ref, idx_ref, loss_ref, m_buf, z2_buf):
    i = pl.program_id(0)
    cur = lax.rem(i, 2)
    prev = lax.rem(i + 1, 2)
    cb = cb_ref[...]
    c2 = jnp.sum(cb * cb, axis=1)[None, :]

    mp = m_buf[prev]
    z2p = z2_buf[prev]
    d = (z2p + c2) - 2.0 * mp
    dmin = jnp.min(d, axis=1, keepdims=True)
    j = lax.broadcasted_iota(jnp.int32, (1, _NUM_EMB), 1).astype(jnp.float32)
    idxf = jnp.min(jnp.where(d == dmin, j, jnp.float32(_NUM_EMB)), axis=1)
    idx_ref[...] = idxf.astype(jnp.int32)

    @pl.when(i == 0)
    def _():
        loss_ref[0, 0] = 0.0

    loss_ref[0, 0] += jnp.where(i > 0, jnp.sum(dmin), 0.0)

    zt = z_ref[...]
    m_buf[cur] = lax.dot_general(zt, cb, (((1,), (1,)), ((), ())),
                                 preferred_element_type=jnp.float32)
    z2_buf[cur] = jnp.sum(zt * zt, axis=1, keepdims=True)


def _make_argmin(rows):
    grid = rows // _TILE
    return pl.pallas_call(
        _dist_body,
        grid=(grid + 1,),
        in_specs=[
            pl.BlockSpec((_TILE, _DIM), lambda i: (jnp.minimum(i, grid - 1), 0)),
            pl.BlockSpec((_NUM_EMB, _DIM), lambda i: (0, 0)),
        ],
        out_specs=[
            pl.BlockSpec((_TILE,), lambda i: (jnp.maximum(i, 1) - 1,)),
            pl.BlockSpec((1, 1), lambda i: (0, 0), memory_space=pltpu.SMEM),
        ],
        out_shape=[
            jax.ShapeDtypeStruct((rows,), jnp.int32),
            jax.ShapeDtypeStruct((1, 1), jnp.float32),
        ],
        scratch_shapes=[
            pltpu.VMEM((2, _TILE, _NUM_EMB), jnp.float32),
            pltpu.VMEM((2, _TILE, 1), jnp.float32),
        ],
    )


_sc_mesh = plsc.VectorSubcoreMesh(core_axis_name="c", subcore_axis_name="s")


def _make_gather(rows):
    bpw = rows // _NW
    chunks = []
    off = 0
    while off < bpw:
        sz = min(_CH, bpw - off)
        chunks.append((off, sz))
        off += sz

    @functools.partial(
        pl.kernel,
        mesh=_sc_mesh,
        out_type=jax.ShapeDtypeStruct((rows, 2 * _DIM), jnp.float32),
        scratch_types=[
            pltpu.VMEM((bpw,), jnp.int32),
            pltpu.VMEM((bpw, 2 * _DIM), jnp.float32),
            pltpu.SemaphoreType.DMA,
        ],
    )
    def _sc_gather(cb_hbm, idx_hbm, out_hbm, idx_v, rows_v, sem):
        wid = lax.axis_index("s") * _NC + lax.axis_index("c")
        base = wid * bpw
        pltpu.sync_copy(idx_hbm.at[pl.ds(base, bpw)], idx_v)
        copies = []
        for off, sz in chunks:
            copies.append(
                pltpu.async_copy(
                    cb_hbm.at[idx_v.at[pl.ds(off, sz)]],
                    rows_v.at[pl.ds(off, sz)],
                    sem,
                ))
        for c in copies:
            c.wait()
        pltpu.sync_copy(rows_v, out_hbm.at[pl.ds(base, bpw)])

    return _sc_gather


_argmin_full = _make_argmin(_ROWS)
_gather_full = _make_gather(_ROWS)


def kernel(z, codebook):
    zz = z[0]
    z_flat = zz.reshape(-1, zz.shape[-1])
    cb_pad = jnp.pad(codebook, ((0, 0), (0, _DIM)))
    idx, loss_sum = _argmin_full(z_flat, codebook)
    zq = _gather_full(cb_pad, idx)[:, :_DIM]
    m = loss_sum[0, 0] / (_ROWS * _DIM)
    vq_loss = m + 0.1 * m
    return zq.reshape(zz.shape), vq_loss
